# baseline (speedup 1.0000x reference)
import functools

import jax
import jax.numpy as jnp
from jax import lax
from jax.experimental import pallas as pl
from jax.experimental.pallas import tpu as pltpu
from jax.experimental.pallas import tpu_sc as plsc

NUM_CORES = 2
NUM_SUBCORES = 16
NW = NUM_CORES * NUM_SUBCORES
LANES = 16
GB = 128


def _make_sc_pool(batch, seq, emb_dim, pad_dim, cb):
    rows_per_w = batch // NW
    chunks_per_w = rows_per_w // cb
    tok_real = cb * seq
    full_bursts = (tok_real + GB - 1) // GB
    mrows = tok_real // 8
    dgroups = emb_dim // LANES
    inv = 1.0 / float(seq)

    mesh = plsc.VectorSubcoreMesh(
        core_axis_name="c", subcore_axis_name="s",
        num_cores=NUM_CORES, num_subcores=NUM_SUBCORES)

    @functools.partial(
        pl.kernel,
        out_type=jax.ShapeDtypeStruct((batch, emb_dim), jnp.float32),
        mesh=mesh,
        scratch_types=[
            pltpu.VMEM((8, GB), jnp.int32),
            pltpu.VMEM((full_bursts * GB, pad_dim), jnp.float32),
            pltpu.VMEM((mrows, GB), jnp.float32),
            pltpu.VMEM((cb, emb_dim), jnp.float32),
            pltpu.SemaphoreType.DMA,
        ],
    )
    def sc_pool(tok_hbm, emb_hbm, mask_hbm, pooled_hbm,
                idx_v, rows_v, mask_v, pooled_v, sem):
        wid = lax.axis_index("s") * NUM_CORES + lax.axis_index("c")

        def accum_row(b):
            base = b * seq

            def tok_body(t, accs):
                r = base + t
                mrow = r // 8
                mlane = (r % 8) * LANES
                m = mask_v[mrow, pl.ds(mlane, LANES)]
                out = []
                for d in range(dgroups):
                    lo = rows_v[r, pl.ds(d * LANES, LANES)]
                    hi = rows_v[r, pl.ds(emb_dim + d * LANES, LANES)]
                    out.append(accs[d] + lo + (hi - lo) * m)
                return tuple(out)

            accs = lax.fori_loop(
                0, seq, tok_body,
                tuple(jnp.zeros((LANES,), jnp.float32)
                      for _ in range(dgroups)),
                unroll=5)
            for d in range(dgroups):
                pooled_v[b, pl.ds(d * LANES, LANES)] = accs[d] * inv

        def chunk_body(c, carry):
            g = wid * chunks_per_w + c
            pltpu.sync_copy(tok_hbm.at[g], idx_v)
            pltpu.sync_copy(mask_hbm.at[g], mask_v)
            copies = [
                pltpu.async_copy(emb_hbm.at[idx_v.at[j]],
                                 rows_v.at[pl.ds(j * GB, GB)], sem)
                for j in range(full_bursts)
            ]
            done = 0
            for j in range(full_bursts):
                copies[j].wait()
                hi = min(cb, (GB * (j + 1) - seq) // seq + 1)
                for b in range(done, hi):
                    accum_row(b)
                done = hi
            for b in range(done, cb):
                accum_row(b)
            pltpu.sync_copy(pooled_v, pooled_hbm.at[pl.ds(g * cb, cb)])
            return carry

        lax.fori_loop(0, chunks_per_w, chunk_body, 0)

    return sc_pool


def _mm_body(x_ref, w_ref, b_ref, o_ref):
    o_ref[...] = (jnp.dot(x_ref[...], w_ref[...],
                          preferred_element_type=jnp.float32)
                  + b_ref[...])


def kernel(token_ids, embedding, W, b):
    batch, seq = token_ids.shape
    vocab, emb_dim = embedding.shape
    out_dim = W.shape[1]
    pad_dim = 2 * emb_dim
    half = vocab // 2

    cb = 16
    tok_real = cb * seq
    slots = 8 * GB
    total_chunks = batch // cb

    e64 = jnp.eye(emb_dim, dtype=jnp.float32)
    z64 = jnp.zeros((emb_dim, emb_dim), jnp.float32)
    eye3 = jnp.stack([jnp.concatenate([e64, z64], axis=1),
                      jnp.concatenate([z64, e64], axis=1)])
    emb_pad = lax.conv_general_dilated(
        embedding[None], eye3, window_strides=(2,), padding='VALID',
        dimension_numbers=('NWC', 'WIO', 'NWC')).reshape(half, pad_dim)

    tok_i32 = token_ids.astype(jnp.int32)
    tok_flat = tok_i32.reshape(total_chunks, tok_real)
    n_pad = slots - tok_real
    pads = jnp.broadcast_to(
        (jnp.arange(n_pad, dtype=jnp.int32) * 4099) % half,
        (total_chunks, n_pad))
    idx2 = jnp.concatenate([tok_flat // 2, pads], axis=1).reshape(
        total_chunks, 8, GB)

    hsel = (tok_flat % 2).astype(jnp.float32)
    mask = jnp.broadcast_to(
        hsel.reshape(total_chunks, tok_real // 8, 8, 1),
        (total_chunks, tok_real // 8, 8, LANES)).reshape(
            total_chunks, tok_real // 8, 8 * LANES)

    sc_pool = _make_sc_pool(batch, seq, emb_dim, pad_dim, cb)
    pooled = sc_pool(idx2, emb_pad, mask)

    bm = 512
    grid = batch // bm
    out = pl.pallas_call(
        _mm_body,
        grid=(grid,),
        in_specs=[
            pl.BlockSpec((bm, emb_dim), lambda i: (i, 0)),
            pl.BlockSpec((emb_dim, out_dim), lambda i: (0, 0)),
            pl.BlockSpec((1, out_dim), lambda i: (0, 0)),
        ],
        out_specs=pl.BlockSpec((bm, out_dim), lambda i: (i, 0)),
        out_shape=jax.ShapeDtypeStruct((batch, out_dim), jnp.float32),
    )(pooled, W, b.reshape(1, out_dim))
    return out


# speedup vs baseline: 6.7793x; 6.7793x over previous
import functools

import jax
import jax.numpy as jnp
from jax import lax
from jax.experimental import pallas as pl
from jax.experimental.pallas import tpu as pltpu
from jax.experimental.pallas import tpu_sc as plsc

NUM_CORES = 2
NUM_SUBCORES = 16
NW = NUM_CORES * NUM_SUBCORES
LANES = 16
GB = 128


def _make_sc_pool(batch, seq, emb_dim, table_rows, cb):
    rows_per_w = batch // NW
    chunks_per_w = rows_per_w // cb
    tok_real = cb * seq
    full_bursts = (tok_real + GB - 1) // GB
    dgroups = emb_dim // LANES
    inv = 1.0 / float(seq)

    mesh = plsc.VectorSubcoreMesh(
        core_axis_name="c", subcore_axis_name="s",
        num_cores=NUM_CORES, num_subcores=NUM_SUBCORES)

    @functools.partial(
        pl.kernel,
        out_type=jax.ShapeDtypeStruct((batch, emb_dim), jnp.float32),
        mesh=mesh,
        scratch_types=[
            pltpu.VMEM((8, GB), jnp.int32),
            pltpu.VMEM((full_bursts * GB, emb_dim), jnp.float32),
            pltpu.VMEM((cb, emb_dim), jnp.float32),
            pltpu.SemaphoreType.DMA,
        ],
        compiler_params=pltpu.CompilerParams(use_tc_tiling_on_sc=False),
    )
    def sc_pool(tok_hbm, emb_hbm, pooled_hbm, idx_v, rows_v, pooled_v, sem):
        wid = lax.axis_index("s") * NUM_CORES + lax.axis_index("c")

        def accum_row(b):
            base = b * seq

            def tok_body(t, accs):
                r = base + t
                return tuple(
                    accs[d] + rows_v[r, pl.ds(d * LANES, LANES)]
                    for d in range(dgroups))

            accs = lax.fori_loop(
                0, seq, tok_body,
                tuple(jnp.zeros((LANES,), jnp.float32)
                      for _ in range(dgroups)),
                unroll=5)
            for d in range(dgroups):
                pooled_v[b, pl.ds(d * LANES, LANES)] = accs[d] * inv

        def chunk_body(c, carry):
            g = wid * chunks_per_w + c
            pltpu.sync_copy(tok_hbm.at[g], idx_v)
            copies = [
                pltpu.async_copy(emb_hbm.at[idx_v.at[j]],
                                 rows_v.at[pl.ds(j * GB, GB)], sem)
                for j in range(full_bursts)
            ]
            done = 0
            for j in range(full_bursts):
                copies[j].wait()
                hi = min(cb, (GB * (j + 1) - seq) // seq + 1)
                for b in range(done, hi):
                    accum_row(b)
                done = hi
            for b in range(done, cb):
                accum_row(b)
            pltpu.sync_copy(pooled_v, pooled_hbm.at[pl.ds(g * cb, cb)])
            return carry

        lax.fori_loop(0, chunks_per_w, chunk_body, 0)

    return sc_pool


def _mm_body(x_ref, w_ref, b_ref, o_ref):
    o_ref[...] = (jnp.dot(x_ref[...], w_ref[...],
                          preferred_element_type=jnp.float32)
                  + b_ref[...])


def kernel(token_ids, embedding, W, b):
    batch, seq = token_ids.shape
    vocab, emb_dim = embedding.shape
    out_dim = W.shape[1]
    pad_dim = 2 * emb_dim

    cb = 16
    tok_real = cb * seq
    slots = 8 * GB
    total_chunks = batch // cb

    eye_pad = jnp.eye(emb_dim, pad_dim, dtype=jnp.float32)
    emb2 = (embedding @ eye_pad).reshape(2 * vocab, emb_dim)

    tok_flat = token_ids.astype(jnp.int32).reshape(total_chunks, tok_real)
    n_pad = slots - tok_real
    pads = jnp.broadcast_to(
        (jnp.arange(n_pad, dtype=jnp.int32) * 4099) % vocab,
        (total_chunks, n_pad))
    idx2 = (jnp.concatenate([tok_flat, pads], axis=1) * 2).reshape(
        total_chunks, 8, GB)

    sc_pool = _make_sc_pool(batch, seq, emb_dim, 2 * vocab, cb)
    pooled = sc_pool(idx2, emb2)

    bm = 512
    grid = batch // bm
    out = pl.pallas_call(
        _mm_body,
        grid=(grid,),
        in_specs=[
            pl.BlockSpec((bm, emb_dim), lambda i: (i, 0)),
            pl.BlockSpec((emb_dim, out_dim), lambda i: (0, 0)),
            pl.BlockSpec((1, out_dim), lambda i: (0, 0)),
        ],
        out_specs=pl.BlockSpec((bm, out_dim), lambda i: (i, 0)),
        out_shape=jax.ShapeDtypeStruct((batch, out_dim), jnp.float32),
    )(pooled, W, b.reshape(1, out_dim))
    return out


# speedup vs baseline: 6.8789x; 1.0147x over previous
import functools

import jax
import jax.numpy as jnp
from jax import lax
from jax.experimental import pallas as pl
from jax.experimental.pallas import tpu as pltpu
from jax.experimental.pallas import tpu_sc as plsc

NUM_CORES = 2
NUM_SUBCORES = 16
NW = NUM_CORES * NUM_SUBCORES
LANES = 16
GB = 128


def _make_sc_pool(batch, seq, emb_dim, table_rows, cb):
    rows_per_w = batch // NW
    chunks_per_w = rows_per_w // cb
    tok_real = cb * seq
    full_bursts = (tok_real + GB - 1) // GB
    dgroups = emb_dim // LANES
    inv = 1.0 / float(seq)

    mesh = plsc.VectorSubcoreMesh(
        core_axis_name="c", subcore_axis_name="s",
        num_cores=NUM_CORES, num_subcores=NUM_SUBCORES)

    @functools.partial(
        pl.kernel,
        out_type=jax.ShapeDtypeStruct((batch, emb_dim), jnp.float32),
        mesh=mesh,
        scratch_types=[
            pltpu.VMEM((8, GB), jnp.int32),
            pltpu.VMEM((8, GB), jnp.int32),
            pltpu.VMEM((full_bursts * GB, emb_dim), jnp.float32),
            pltpu.VMEM((full_bursts * GB, emb_dim), jnp.float32),
            pltpu.VMEM((cb, emb_dim), jnp.float32),
            pltpu.SemaphoreType.DMA,
            pltpu.SemaphoreType.DMA,
        ],
        compiler_params=pltpu.CompilerParams(use_tc_tiling_on_sc=False),
    )
    def sc_pool(tok_hbm, emb_hbm, pooled_hbm,
                idx_a, idx_b, rows_a, rows_b, pooled_v, sem_a, sem_b):
        wid = lax.axis_index("s") * NUM_CORES + lax.axis_index("c")
        gbase = wid * chunks_per_w

        def fire(c, idx_v, rows_v, sem):
            pltpu.sync_copy(tok_hbm.at[gbase + c], idx_v)
            for j in range(full_bursts):
                pltpu.async_copy(emb_hbm.at[idx_v.at[j]],
                                 rows_v.at[pl.ds(j * GB, GB)], sem)

        def accum_row(rows_v, b):
            base = b * seq

            def tok_body(t, accs):
                r = base + t
                return tuple(
                    accs[d] + rows_v[r, pl.ds(d * LANES, LANES)]
                    for d in range(dgroups))

            accs = lax.fori_loop(
                0, seq, tok_body,
                tuple(jnp.zeros((LANES,), jnp.float32)
                      for _ in range(dgroups)),
                unroll=5)
            for d in range(dgroups):
                pooled_v[b, pl.ds(d * LANES, LANES)] = accs[d] * inv

        def drain_accum(c, idx_v, rows_v, sem):
            done = 0
            for j in range(full_bursts):
                pltpu.make_async_copy(
                    emb_hbm.at[idx_v.at[j]],
                    rows_v.at[pl.ds(j * GB, GB)], sem).wait()
                hi = min(cb, (GB * (j + 1) - seq) // seq + 1)
                for b in range(done, hi):
                    accum_row(rows_v, b)
                done = hi
            for b in range(done, cb):
                accum_row(rows_v, b)
            pltpu.sync_copy(pooled_v,
                            pooled_hbm.at[pl.ds((gbase + c) * cb, cb)])

        fire(0, idx_a, rows_a, sem_a)

        def pair_body(i, carry):
            ca = 2 * i
            fire(ca + 1, idx_b, rows_b, sem_b)
            drain_accum(ca, idx_a, rows_a, sem_a)

            @pl.when(ca + 2 < chunks_per_w)
            def _():
                fire(ca + 2, idx_a, rows_a, sem_a)

            drain_accum(ca + 1, idx_b, rows_b, sem_b)
            return carry

        lax.fori_loop(0, chunks_per_w // 2, pair_body, 0)

    return sc_pool


def _mm_body(x_ref, w_ref, b_ref, o_ref):
    o_ref[...] = (jnp.dot(x_ref[...], w_ref[...],
                          preferred_element_type=jnp.float32)
                  + b_ref[...])


def kernel(token_ids, embedding, W, b):
    batch, seq = token_ids.shape
    vocab, emb_dim = embedding.shape
    out_dim = W.shape[1]
    pad_dim = 2 * emb_dim

    cb = 16
    tok_real = cb * seq
    slots = 8 * GB
    total_chunks = batch // cb

    eye_pad = jnp.eye(emb_dim, pad_dim, dtype=jnp.float32)
    emb2 = (embedding @ eye_pad).reshape(2 * vocab, emb_dim)

    tok_flat = token_ids.astype(jnp.int32).reshape(total_chunks, tok_real)
    n_pad = slots - tok_real
    pads = jnp.broadcast_to(
        (jnp.arange(n_pad, dtype=jnp.int32) * 4099) % vocab,
        (total_chunks, n_pad))
    idx2 = (jnp.concatenate([tok_flat, pads], axis=1) * 2).reshape(
        total_chunks, 8, GB)

    sc_pool = _make_sc_pool(batch, seq, emb_dim, 2 * vocab, cb)
    pooled = sc_pool(idx2, emb2)

    bm = 512
    grid = batch // bm
    out = pl.pallas_call(
        _mm_body,
        grid=(grid,),
        in_specs=[
            pl.BlockSpec((bm, emb_dim), lambda i: (i, 0)),
            pl.BlockSpec((emb_dim, out_dim), lambda i: (0, 0)),
            pl.BlockSpec((1, out_dim), lambda i: (0, 0)),
        ],
        out_specs=pl.BlockSpec((bm, out_dim), lambda i: (i, 0)),
        out_shape=jax.ShapeDtypeStruct((batch, out_dim), jnp.float32),
    )(pooled, W, b.reshape(1, out_dim))
    return out


# speedup vs baseline: 6.9425x; 1.0092x over previous
import functools

import jax
import jax.numpy as jnp
from jax import lax
from jax.experimental import pallas as pl
from jax.experimental.pallas import tpu as pltpu
from jax.experimental.pallas import tpu_sc as plsc

NUM_CORES = 2
NUM_SUBCORES = 16
NW = NUM_CORES * NUM_SUBCORES
LANES = 16
GB = 128


def _make_sc_pool(batch, seq, emb_dim, table_rows, cb):
    rows_per_w = batch // NW
    chunks_per_w = rows_per_w // cb
    tok_real = cb * seq
    full_bursts = (tok_real + GB - 1) // GB
    dgroups = emb_dim // LANES
    inv = 1.0 / float(seq)

    mesh = plsc.VectorSubcoreMesh(
        core_axis_name="c", subcore_axis_name="s",
        num_cores=NUM_CORES, num_subcores=NUM_SUBCORES)

    @functools.partial(
        pl.kernel,
        out_type=jax.ShapeDtypeStruct((batch, 2 * emb_dim), jnp.float32),
        mesh=mesh,
        scratch_types=[
            pltpu.VMEM((8, GB), jnp.int32),
            pltpu.VMEM((8, GB), jnp.int32),
            pltpu.VMEM((full_bursts * GB, emb_dim), jnp.float32),
            pltpu.VMEM((full_bursts * GB, emb_dim), jnp.float32),
            pltpu.VMEM((cb, emb_dim), jnp.float32),
            pltpu.SemaphoreType.DMA,
            pltpu.SemaphoreType.DMA,
        ],
        compiler_params=pltpu.CompilerParams(use_tc_tiling_on_sc=False),
    )
    def sc_pool(tok_hbm, emb_hbm, pooled_hbm,
                idx_a, idx_b, rows_a, rows_b, pooled_v, sem_a, sem_b):
        wid = lax.axis_index("s") * NUM_CORES + lax.axis_index("c")
        gbase = wid * chunks_per_w

        def fire(c, idx_v, rows_v, sem):
            pltpu.sync_copy(tok_hbm.at[gbase + c], idx_v)
            for j in range(full_bursts):
                pltpu.async_copy(emb_hbm.at[idx_v.at[j]],
                                 rows_v.at[pl.ds(j * GB, GB)], sem)

        def accum_row(rows_v, b):
            base = b * seq

            def tok_body(t, accs):
                r = base + t
                return tuple(
                    accs[d] + rows_v[r, pl.ds(d * LANES, LANES)]
                    for d in range(dgroups))

            accs = lax.fori_loop(
                0, seq, tok_body,
                tuple(jnp.zeros((LANES,), jnp.float32)
                      for _ in range(dgroups)),
                unroll=5)
            for d in range(dgroups):
                pooled_v[b, pl.ds(d * LANES, LANES)] = accs[d] * inv

        def drain_accum(c, idx_v, rows_v, sem):
            done = 0
            for j in range(full_bursts):
                pltpu.make_async_copy(
                    emb_hbm.at[idx_v.at[j]],
                    rows_v.at[pl.ds(j * GB, GB)], sem).wait()
                hi = min(cb, (GB * (j + 1) - seq) // seq + 1)
                for b in range(done, hi):
                    accum_row(rows_v, b)
                done = hi
            for b in range(done, cb):
                accum_row(rows_v, b)
            pltpu.sync_copy(
                pooled_v,
                pooled_hbm.at[pl.ds((gbase + c) * cb, cb),
                              pl.ds(0, emb_dim)])

        fire(0, idx_a, rows_a, sem_a)

        def pair_body(i, carry):
            ca = 2 * i
            fire(ca + 1, idx_b, rows_b, sem_b)
            drain_accum(ca, idx_a, rows_a, sem_a)

            @pl.when(ca + 2 < chunks_per_w)
            def _():
                fire(ca + 2, idx_a, rows_a, sem_a)

            drain_accum(ca + 1, idx_b, rows_b, sem_b)
            return carry

        lax.fori_loop(0, chunks_per_w // 2, pair_body, 0)

    return sc_pool


def _mm_body(x_ref, w_ref, b_ref, o_ref):
    emb_dim = w_ref.shape[0]
    o_ref[...] = (jnp.dot(x_ref[:, :emb_dim], w_ref[...],
                          preferred_element_type=jnp.float32)
                  + b_ref[...])


def kernel(token_ids, embedding, W, b):
    batch, seq = token_ids.shape
    vocab, emb_dim = embedding.shape
    out_dim = W.shape[1]
    pad_dim = 2 * emb_dim

    cb = 16
    tok_real = cb * seq
    slots = 8 * GB
    total_chunks = batch // cb

    eye_pad = jnp.eye(emb_dim, pad_dim, dtype=jnp.float32)
    emb2 = (embedding @ eye_pad).reshape(2 * vocab, emb_dim)

    tok_flat = token_ids.astype(jnp.int32).reshape(total_chunks, tok_real)
    n_pad = slots - tok_real
    pads = jnp.broadcast_to(
        (jnp.arange(n_pad, dtype=jnp.int32) * 4099) % vocab,
        (total_chunks, n_pad))
    idx2 = (jnp.concatenate([tok_flat, pads], axis=1) * 2).reshape(
        total_chunks, 8, GB)

    sc_pool = _make_sc_pool(batch, seq, emb_dim, 2 * vocab, cb)
    pooled = sc_pool(idx2, emb2)

    bm = 512
    grid = batch // bm
    out = pl.pallas_call(
        _mm_body,
        grid=(grid,),
        in_specs=[
            pl.BlockSpec((bm, 2 * emb_dim), lambda i: (i, 0)),
            pl.BlockSpec((emb_dim, out_dim), lambda i: (0, 0)),
            pl.BlockSpec((1, out_dim), lambda i: (0, 0)),
        ],
        out_specs=pl.BlockSpec((bm, out_dim), lambda i: (i, 0)),
        out_shape=jax.ShapeDtypeStruct((batch, out_dim), jnp.float32),
    )(pooled, W, b.reshape(1, out_dim))
    return out


# speedup vs baseline: 7.0393x; 1.0140x over previous
import functools

import jax
import jax.numpy as jnp
from jax import lax
from jax.experimental import pallas as pl
from jax.experimental.pallas import tpu as pltpu
from jax.experimental.pallas import tpu_sc as plsc

NUM_CORES = 2
NUM_SUBCORES = 16
NW = NUM_CORES * NUM_SUBCORES
LANES = 16
GB = 128


def _make_sc_pool(batch, seq, emb_dim, table_rows, cb):
    rows_per_w = batch // NW
    chunks_per_w = rows_per_w // cb
    tok_real = cb * seq
    full_bursts = (tok_real + GB - 1) // GB
    dgroups = emb_dim // LANES
    inv = 1.0 / float(seq)

    mesh = plsc.VectorSubcoreMesh(
        core_axis_name="c", subcore_axis_name="s",
        num_cores=NUM_CORES, num_subcores=NUM_SUBCORES)

    @functools.partial(
        pl.kernel,
        out_type=jax.ShapeDtypeStruct((batch, 2 * emb_dim), jnp.float32),
        mesh=mesh,
        scratch_types=[
            pltpu.VMEM((8, GB), jnp.int32),
            pltpu.VMEM((8, GB), jnp.int32),
            pltpu.VMEM((full_bursts * GB, emb_dim), jnp.float32),
            pltpu.VMEM((full_bursts * GB, emb_dim), jnp.float32),
            pltpu.VMEM((cb, emb_dim), jnp.float32),
            pltpu.SemaphoreType.DMA,
            pltpu.SemaphoreType.DMA,
        ],
        compiler_params=pltpu.CompilerParams(use_tc_tiling_on_sc=False),
    )
    def sc_pool(tok_hbm, emb_hbm, pooled_hbm,
                idx_a, idx_b, rows_a, rows_b, pooled_v, sem_a, sem_b):
        wid = lax.axis_index("s") * NUM_CORES + lax.axis_index("c")
        gbase = wid * chunks_per_w

        def fire(c, idx_v, rows_v, sem):
            pltpu.sync_copy(tok_hbm.at[gbase + c], idx_v)
            for j in range(full_bursts):
                pltpu.async_copy(emb_hbm.at[idx_v.at[j]],
                                 rows_v.at[pl.ds(j * GB, GB)], sem)

        def accum_row(rows_v, b):
            base = b * seq

            def tok_body(t, accs):
                r = base + t
                return tuple(
                    accs[d] + rows_v[r, pl.ds(d * LANES, LANES)]
                    for d in range(dgroups))

            accs = lax.fori_loop(
                0, seq, tok_body,
                tuple(jnp.zeros((LANES,), jnp.float32)
                      for _ in range(dgroups)),
                unroll=5)
            for d in range(dgroups):
                pooled_v[b, pl.ds(d * LANES, LANES)] = accs[d] * inv

        def drain_accum(c, idx_v, rows_v, sem):
            done = 0
            for j in range(full_bursts):
                pltpu.make_async_copy(
                    emb_hbm.at[idx_v.at[j]],
                    rows_v.at[pl.ds(j * GB, GB)], sem).wait()
                hi = min(cb, (GB * (j + 1) - seq) // seq + 1)
                for b in range(done, hi):
                    accum_row(rows_v, b)
                done = hi
            for b in range(done, cb):
                accum_row(rows_v, b)
            pltpu.sync_copy(
                pooled_v,
                pooled_hbm.at[pl.ds((gbase + c) * cb, cb),
                              pl.ds(0, emb_dim)])

        fire(0, idx_a, rows_a, sem_a)

        def pair_body(i, carry):
            ca = 2 * i
            fire(ca + 1, idx_b, rows_b, sem_b)
            drain_accum(ca, idx_a, rows_a, sem_a)

            @pl.when(ca + 2 < chunks_per_w)
            def _():
                fire(ca + 2, idx_a, rows_a, sem_a)

            drain_accum(ca + 1, idx_b, rows_b, sem_b)
            return carry

        lax.fori_loop(0, chunks_per_w // 2, pair_body, 0)

    return sc_pool


def _mm_body(x_ref, w_ref, b_ref, o_ref):
    emb_dim = w_ref.shape[0]
    o_ref[...] = (jnp.dot(x_ref[:, :emb_dim], w_ref[...],
                          preferred_element_type=jnp.float32)
                  + b_ref[...])


def kernel(token_ids, embedding, W, b):
    batch, seq = token_ids.shape
    vocab, emb_dim = embedding.shape
    out_dim = W.shape[1]
    pad_dim = 2 * emb_dim

    cb = 16
    tok_real = cb * seq
    slots = 8 * GB
    total_chunks = batch // cb

    eye_pad = jnp.eye(emb_dim, pad_dim, dtype=jnp.float32)
    emb2 = (embedding @ eye_pad).reshape(2 * vocab, emb_dim)

    tok_flat = token_ids.astype(jnp.int32).reshape(total_chunks, tok_real)
    n_pad = slots - tok_real
    pads = jnp.broadcast_to(
        (jnp.arange(n_pad, dtype=jnp.int32) * 4099) % vocab,
        (total_chunks, n_pad))
    idx2 = (jnp.concatenate([tok_flat, pads], axis=1) * 2).reshape(
        total_chunks, 8, GB)

    sc_pool = _make_sc_pool(batch, seq, emb_dim, 2 * vocab, cb)
    pooled = sc_pool(idx2, emb2)

    bm = 2048
    grid = batch // bm
    out = pl.pallas_call(
        _mm_body,
        grid=(grid,),
        in_specs=[
            pl.BlockSpec((bm, 2 * emb_dim), lambda i: (i, 0)),
            pl.BlockSpec((emb_dim, out_dim), lambda i: (0, 0)),
            pl.BlockSpec((1, out_dim), lambda i: (0, 0)),
        ],
        out_specs=pl.BlockSpec((bm, out_dim), lambda i: (i, 0)),
        out_shape=jax.ShapeDtypeStruct((batch, out_dim), jnp.float32),
    )(pooled, W, b.reshape(1, out_dim))
    return out
